# Initial kernel scaffold; baseline (speedup 1.0000x reference)
import jax
import jax.numpy as jnp
from jax.experimental import pallas as pl


def kernel(inputs, table):
    raise NotImplementedError("write your pallas kernel here")


# speedup vs baseline: 4.7507x; 4.7507x over previous
import jax
import jax.numpy as jnp
from jax.experimental import pallas as pl

_B = 4
_S = 8192
_D = 1024
_RBLK = 256


def _copy_kernel(table_ref, out_ref):
    out_ref[...] = jnp.broadcast_to(table_ref[...][None, :, :], (_B, _RBLK, _D))


def kernel(inputs, table):
    del inputs
    grid = (_S // _RBLK,)
    return pl.pallas_call(
        _copy_kernel,
        grid=grid,
        in_specs=[pl.BlockSpec((_RBLK, _D), lambda r: (r, 0))],
        out_specs=pl.BlockSpec((_B, _RBLK, _D), lambda r: (0, r, 0)),
        out_shape=jax.ShapeDtypeStruct((_B, _S, _D), table.dtype),
    )(table)


# speedup vs baseline: 5.0420x; 1.0613x over previous
import jax
import jax.numpy as jnp
from jax.experimental import pallas as pl

_B = 4
_S = 8192
_D = 1024
_RBLK = 512


def _copy_kernel(table_ref, out_ref):
    out_ref[...] = jnp.broadcast_to(table_ref[...][None, :, :], (_B, _RBLK, _D))


def kernel(inputs, table):
    del inputs
    grid = (_S // _RBLK,)
    return pl.pallas_call(
        _copy_kernel,
        grid=grid,
        in_specs=[pl.BlockSpec((_RBLK, _D), lambda r: (r, 0))],
        out_specs=pl.BlockSpec((_B, _RBLK, _D), lambda r: (0, r, 0)),
        out_shape=jax.ShapeDtypeStruct((_B, _S, _D), table.dtype),
    )(table)


# speedup vs baseline: 5.1785x; 1.0271x over previous
import jax
import jax.numpy as jnp
from jax.experimental import pallas as pl

_B = 4
_S = 8192
_D = 1024
_RBLK = 1024


def _copy_kernel(table_ref, out_ref):
    out_ref[...] = jnp.broadcast_to(table_ref[...][None, :, :], (_B, _RBLK, _D))


def kernel(inputs, table):
    del inputs
    grid = (_S // _RBLK,)
    return pl.pallas_call(
        _copy_kernel,
        grid=grid,
        in_specs=[pl.BlockSpec((_RBLK, _D), lambda r: (r, 0))],
        out_specs=pl.BlockSpec((_B, _RBLK, _D), lambda r: (0, r, 0)),
        out_shape=jax.ShapeDtypeStruct((_B, _S, _D), table.dtype),
    )(table)
